# Initial kernel scaffold; baseline (speedup 1.0000x reference)
import jax
import jax.numpy as jnp
from jax.experimental import pallas as pl


def kernel(features_white, features_black, is_white_stm, W_white, b_white, W_black, b_black, W_out, b_out):
    raise NotImplementedError("write your pallas kernel here")


# speedup vs baseline: 2.2963x; 2.2963x over previous
import jax
import jax.numpy as jnp
from jax import lax
from jax.experimental import pallas as pl
from jax.experimental.pallas import tpu as pltpu, tpu_sc as plsc

NUM_FEATURES = 6144
HIDDEN = 1024
B = 16384
F = 32

_NC = 2
_NS = 16
_NW = _NC * _NS
_BPW = B // _NW
_ROWS = 2 * F


def _body(wext, idx_hbm, stm_hbm, bias_hbm, wout_hbm, params_hbm, out_hbm,
          idx_v, buf, stm_v, bias_v, wout_v, params_v, out_v, sem):
    wid = lax.axis_index("s") * _NC + lax.axis_index("c")
    base = wid * _BPW

    pltpu.sync_copy(idx_hbm.at[pl.ds(base * _ROWS, _BPW * _ROWS)], idx_v)
    pltpu.sync_copy(stm_hbm.at[pl.ds(base * 16, _BPW * 16)], stm_v)
    pltpu.sync_copy(bias_hbm, bias_v)
    pltpu.sync_copy(wout_hbm, wout_v)
    pltpu.sync_copy(params_hbm, params_v)

    b_out_s = params_v[pl.ds(0, 16)][0]
    lane = lax.iota(jnp.int32, 16)

    def per_batch(j, out_vec):
        pltpu.async_copy(wext.at[idx_v.at[pl.ds(j * _ROWS, _ROWS)]], buf, sem).wait()

        def per_chunk(c, carry):
            d11, d12, d21, d22 = carry
            o = c * 16
            vw = bias_v[pl.ds(o, 16)]
            vb = bias_v[pl.ds(HIDDEN + o, 16)]
            for r in range(F):
                vw = vw + buf[r, pl.ds(o, 16)]
            for r in range(F, 2 * F):
                vb = vb + buf[r, pl.ds(o, 16)]
            aw = jnp.clip(vw, 0.0, 1.0)
            ab = jnp.clip(vb, 0.0, 1.0)
            aw = aw * aw
            ab = ab * ab
            w1 = wout_v[pl.ds(o, 16)]
            w2 = wout_v[pl.ds(HIDDEN + o, 16)]
            return (d11 + aw * w1, d12 + ab * w2, d21 + aw * w2, d22 + ab * w1)

        zero = jnp.zeros((16,), jnp.float32)
        d11, d12, d21, d22 = lax.fori_loop(
            0, HIDDEN // 16, per_chunk, (zero, zero, zero, zero))
        sf = stm_v[pl.ds(j * 16, 16)].astype(jnp.float32)
        dvec = (d11 + d12) * sf + (d21 + d22) * (1.0 - sf)
        tot = b_out_s
        for k in range(16):
            tot = tot + dvec[k]
        out_vec = jnp.where(lane == (j & 15), tot, out_vec)

        @pl.when((j & 15) == 15)
        def _():
            out_v[pl.ds(j - 15, 16)] = out_vec

        return out_vec

    lax.fori_loop(0, _BPW, per_batch, jnp.zeros((16,), jnp.float32))
    pltpu.sync_copy(out_v, out_hbm.at[pl.ds(base, _BPW)])


def kernel(features_white, features_black, is_white_stm,
           W_white, b_white, W_black, b_black, W_out, b_out):
    wext = jnp.concatenate([W_white, W_black], axis=0)
    idx = jnp.concatenate(
        [features_white, features_black + NUM_FEATURES], axis=1).reshape(-1)
    stm = jnp.broadcast_to(
        is_white_stm.astype(jnp.int32).reshape(B, 1), (B, 16)).reshape(-1)
    bias = jnp.concatenate([b_white, b_black])
    wout = W_out.reshape(2 * HIDDEN)
    params = jnp.pad(b_out, (0, 15))

    mesh = plsc.VectorSubcoreMesh(core_axis_name="c", subcore_axis_name="s")
    run = pl.kernel(
        _body,
        out_type=jax.ShapeDtypeStruct((B,), jnp.float32),
        mesh=mesh,
        scratch_types=[
            pltpu.VMEM((_BPW * _ROWS,), jnp.int32),
            pltpu.VMEM((_ROWS, HIDDEN), jnp.float32),
            pltpu.VMEM((_BPW * 16,), jnp.int32),
            pltpu.VMEM((2 * HIDDEN,), jnp.float32),
            pltpu.VMEM((2 * HIDDEN,), jnp.float32),
            pltpu.VMEM((16,), jnp.float32),
            pltpu.VMEM((_BPW,), jnp.float32),
            pltpu.SemaphoreType.DMA,
        ],
    )
    out = run(wext, idx, stm, bias, wout, params)
    return out.reshape(B, 1)


# speedup vs baseline: 3.2415x; 1.4116x over previous
import jax
import jax.numpy as jnp
from jax import lax
from jax.experimental import pallas as pl
from jax.experimental.pallas import tpu as pltpu, tpu_sc as plsc

NUM_FEATURES = 6144
HIDDEN = 1024
B = 16384
F = 32

_NC = 2
_NS = 16
_NW = _NC * _NS
_BPW = B // _NW
_ROWS = 2 * F
_RW = HIDDEN // 2


def _body(wext, idx_hbm, stm_hbm, bias_hbm, wout_hbm, params_hbm, out_hbm,
          idx_v, buf0, buf1, stm_v, bias_v, wout_v, params_v, out_v,
          sem0, sem1):
    wid = lax.axis_index("s") * _NC + lax.axis_index("c")
    base = wid * _BPW

    pltpu.sync_copy(idx_hbm.at[pl.ds(base * _ROWS, _BPW * _ROWS)], idx_v)
    pltpu.sync_copy(stm_hbm.at[pl.ds(base * 16, _BPW * 16)], stm_v)
    pltpu.sync_copy(bias_hbm, bias_v)
    pltpu.sync_copy(wout_hbm, wout_v)
    pltpu.sync_copy(params_hbm, params_v)

    b_out_s = params_v[pl.ds(0, 16)][0]
    lane = lax.iota(jnp.int32, 16)
    himask = jnp.uint32(0xFFFF0000)

    def start_gather(j, buf, sem):
        pltpu.async_copy(wext.at[idx_v.at[pl.ds(j * _ROWS, _ROWS)]], buf, sem)

    def wait_gather(buf, sem):
        pltpu.make_async_copy(wext.at[pl.ds(0, _ROWS)], buf, sem).wait()

    def compute(j, buf, out_vec):
        def per_chunk(c, carry):
            d11, d12, d21, d22 = carry
            o = c * 16

            def side(row0, boff):
                a0 = jnp.zeros((16,), jnp.float32)
                a1 = jnp.zeros((16,), jnp.float32)
                for r in range(row0, row0 + F):
                    u = buf[r, pl.ds(o, 16)].astype(jnp.uint32)
                    a0 = a0 + plsc.bitcast(u << 16, jnp.float32)
                    a1 = a1 + plsc.bitcast(u & himask, jnp.float32)
                ub = bias_v[pl.ds(boff + o, 16)].astype(jnp.uint32)
                a0 = a0 + plsc.bitcast(ub << 16, jnp.float32)
                a1 = a1 + plsc.bitcast(ub & himask, jnp.float32)
                a0 = jnp.clip(a0, 0.0, 1.0)
                a1 = jnp.clip(a1, 0.0, 1.0)
                return a0 * a0, a1 * a1

            aw0, aw1 = side(0, 0)
            ab0, ab1 = side(F, _RW)
            o32 = c * 32
            w10 = wout_v[pl.ds(o32, 16)]
            w11 = wout_v[pl.ds(o32 + 16, 16)]
            w20 = wout_v[pl.ds(HIDDEN + o32, 16)]
            w21 = wout_v[pl.ds(HIDDEN + o32 + 16, 16)]
            d11 = d11 + aw0 * w10 + aw1 * w11
            d12 = d12 + ab0 * w20 + ab1 * w21
            d21 = d21 + aw0 * w20 + aw1 * w21
            d22 = d22 + ab0 * w10 + ab1 * w11
            return (d11, d12, d21, d22)

        zero = jnp.zeros((16,), jnp.float32)
        d11, d12, d21, d22 = lax.fori_loop(
            0, HIDDEN // 32, per_chunk, (zero, zero, zero, zero))
        sf = stm_v[pl.ds(j * 16, 16)].astype(jnp.float32)
        dvec = (d11 + d12) * sf + (d21 + d22) * (1.0 - sf)
        tot = b_out_s
        for k in range(16):
            tot = tot + dvec[k]
        out_vec = jnp.where(lane == (j & 15), tot, out_vec)

        @pl.when((j & 15) == 15)
        def _():
            out_v[pl.ds(j - 15, 16)] = out_vec

        return out_vec

    start_gather(0, buf0, sem0)

    def per_pair(m, out_vec):
        j0 = 2 * m
        wait_gather(buf0, sem0)
        start_gather(j0 + 1, buf1, sem1)
        out_vec = compute(j0, buf0, out_vec)
        wait_gather(buf1, sem1)

        @pl.when(m < _BPW // 2 - 1)
        def _():
            start_gather(j0 + 2, buf0, sem0)

        return compute(j0 + 1, buf1, out_vec)

    lax.fori_loop(0, _BPW // 2, per_pair, jnp.zeros((16,), jnp.float32))
    pltpu.sync_copy(out_v, out_hbm.at[pl.ds(base, _BPW)])


def _pack_bf16(x):
    xb = x.astype(jnp.bfloat16)
    return jax.lax.bitcast_convert_type(
        xb.reshape(*xb.shape[:-1], xb.shape[-1] // 2, 2), jnp.int32)


def kernel(features_white, features_black, is_white_stm,
           W_white, b_white, W_black, b_black, W_out, b_out):
    wext = _pack_bf16(jnp.concatenate([W_white, W_black], axis=0))
    idx = jnp.concatenate(
        [features_white, features_black + NUM_FEATURES], axis=1).reshape(-1)
    stm = jnp.broadcast_to(
        is_white_stm.astype(jnp.int32).reshape(B, 1), (B, 16)).reshape(-1)
    bias = _pack_bf16(jnp.concatenate([b_white, b_black]))
    wout = W_out.reshape(64, 16, 2).transpose(0, 2, 1).reshape(2 * HIDDEN)
    params = jnp.pad(b_out, (0, 15))

    mesh = plsc.VectorSubcoreMesh(core_axis_name="c", subcore_axis_name="s")
    run = pl.kernel(
        _body,
        out_type=jax.ShapeDtypeStruct((B,), jnp.float32),
        mesh=mesh,
        compiler_params=pltpu.CompilerParams(needs_layout_passes=False),
        scratch_types=[
            pltpu.VMEM((_BPW * _ROWS,), jnp.int32),
            pltpu.VMEM((_ROWS, _RW), jnp.int32),
            pltpu.VMEM((_ROWS, _RW), jnp.int32),
            pltpu.VMEM((_BPW * 16,), jnp.int32),
            pltpu.VMEM((2 * _RW,), jnp.int32),
            pltpu.VMEM((2 * HIDDEN,), jnp.float32),
            pltpu.VMEM((16,), jnp.float32),
            pltpu.VMEM((_BPW,), jnp.float32),
            pltpu.SemaphoreType.DMA,
            pltpu.SemaphoreType.DMA,
        ],
    )
    out = run(wext, idx, stm, bias, wout, params)
    return out.reshape(B, 1)
